# baseline (speedup 1.0000x reference)
import functools

import jax
import jax.numpy as jnp
from jax import lax
from jax.experimental import pallas as pl
from jax.experimental.pallas import tpu as pltpu
from jax.experimental.pallas import tpu_sc as plsc

_SPARSE_SHAPE = (32, 1000, 1000)
_WINDOW_SHAPE = (13, 13, 32)
_GROUP = 4096
_DIM = 128
_N = 65536
_NG = _N // _GROUP


def _window_keys(coords):
    import numpy as np
    sz, sy, sx = _SPARSE_SHAPE
    wx, wy, wz = _WINDOW_SHAPE
    max_x = int(np.ceil(sx / wx) + 1)
    max_y = int(np.ceil(sy / wy) + 1)
    max_z = int(np.ceil(sz / wz) + 1)
    m_per = max_x * max_y * max_z
    x = coords[:, 3]
    y = coords[:, 2]
    z = coords[:, 1]
    win_x = x // wx
    win_y = y // wy
    win_z = z // wz
    cix = x % wx
    ciy = y % wy
    ciz = z % wz
    bwx = coords[:, 0] * m_per + win_x * max_y * max_z + win_y * max_z + win_z
    bwy = coords[:, 0] * m_per + win_y * max_x * max_z + win_x * max_z + win_z
    wvol = wx * wy * wz
    vx = bwx * wvol + cix * (wy * wz) + ciy * wz + ciz
    vy = bwy * wvol + ciy * (wx * wz) + cix * wz + ciz
    return vx, vy


def _elu1(x):
    return jnp.where(x > 0, x + 1.0, jnp.exp(jnp.minimum(x, 0.0)))


def _group_body(x_ref, wqkv_ref, wo_ref, o_ref):
    x = x_ref[...]
    qkv = jnp.dot(x, wqkv_ref[...], preferred_element_type=jnp.float32)
    q = _elu1(qkv[:, :_DIM])
    k = _elu1(qkv[:, _DIM:2 * _DIM])
    v = qkv[:, 2 * _DIM:]
    vaug = jnp.concatenate([v, jnp.ones_like(v)], axis=1)
    s_aug = jax.lax.dot_general(k, vaug, (((0,), (0,)), ((), ())),
                                preferred_element_type=jnp.float32)
    nd = jnp.dot(q, s_aug, preferred_element_type=jnp.float32)
    out = nd[:, :_DIM] / (nd[:, _DIM:] + 1e-6)
    o_ref[...] = x + jnp.dot(out, wo_ref[...], preferred_element_type=jnp.float32)


@functools.partial(jax.jit, static_argnames=())
def _group_op(sorted_x, wqkv, wo):
    return pl.pallas_call(
        _group_body,
        grid=(_NG,),
        in_specs=[
            pl.BlockSpec((_GROUP, _DIM), lambda i: (i, 0)),
            pl.BlockSpec((_DIM, 3 * _DIM), lambda i: (0, 0)),
            pl.BlockSpec((_DIM, _DIM), lambda i: (0, 0)),
        ],
        out_specs=pl.BlockSpec((_GROUP, _DIM), lambda i: (i, 0)),
        out_shape=jax.ShapeDtypeStruct((_N, _DIM), jnp.float32),
    )(sorted_x, wqkv, wo)


_NW = 16
_CNK = _N // _NW
_NV = _CNK // 16
_BINS = 8192
_BPW = _BINS // _NW
_ROWS = _CNK // 128


def _pass_phases(digit_of, load_keys, load_idxs, emit_scatter,
                 refs, wid, off):
    (keys_v, idxs_v, hist_v, woff_v, grid_v, cum_v, base_v,
     sm_v, buf_v, pos2_v, posi_v, kv2_v, iv2_v, ivi_v,
     hist, woff, smalls) = refs
    iota = lax.iota(jnp.int32, 16)

    def zero(ref, nv):
        def zb(i, _):
            ref[pl.ds(i * 16, 16)] = jnp.zeros((16,), jnp.int32)
            return 0
        lax.fori_loop(0, nv, zb, 0)

    load_keys()
    load_idxs()
    zero(hist_v, _BINS // 16)

    def hist_loop(i, _):
        d = digit_of(keys_v[pl.ds(i * 16, 16)])
        cnt, msk = plsc.scan_count(d)
        plsc.addupdate_scatter(hist_v, [d], cnt, mask=msk)
        return 0
    lax.fori_loop(0, _NV, hist_loop, 0)
    pltpu.sync_copy(hist_v, hist.at[wid])
    plsc.subcore_barrier()

    b0 = wid * _BPW
    pltpu.sync_copy(hist.at[:, pl.ds(b0, _BPW)], grid_v)
    zero(cum_v, _BPW // 16)

    def w_loop(w, _):
        def j_loop(j, _):
            s = j * 16
            cv = cum_v[pl.ds(s, 16)]
            woff_v[pl.ds(w * _BPW + s, 16)] = cv
            cum_v[pl.ds(s, 16)] = cv + grid_v[w, pl.ds(s, 16)]
            return 0
        return lax.fori_loop(0, _BPW // 16, j_loop, 0)
    lax.fori_loop(0, _NW, w_loop, 0)

    def sum_loop(j, acc):
        return acc + jnp.sum(cum_v[pl.ds(j * 16, 16)])
    s_b = lax.fori_loop(0, _BPW // 16, sum_loop, jnp.int32(0))
    buf_v[...] = jnp.zeros((16,), jnp.int32) + s_b
    pltpu.sync_copy(buf_v, smalls.at[wid])
    plsc.subcore_barrier()
    pltpu.sync_copy(smalls, sm_v)
    svec = plsc.load_gather(sm_v, [iota, jnp.zeros((16,), jnp.int32)])
    csum = plsc.cumsum(svec)
    sb_excl = jnp.sum(jnp.where(iota == wid, csum - svec, 0))

    def bb_loop(j, carry):
        t = cum_v[pl.ds(j * 16, 16)]
        cs = plsc.cumsum(t)
        base_v[pl.ds(j * 16, 16)] = cs - t + carry
        return carry + jnp.sum(t)
    lax.fori_loop(0, _BPW // 16, bb_loop, sb_excl)

    def wr_loop(w, _):
        def j2(j, _):
            s = j * 16
            woff_v[pl.ds(w * _BPW + s, 16)] = (
                woff_v[pl.ds(w * _BPW + s, 16)] + base_v[pl.ds(s, 16)])
            return 0
        lax.fori_loop(0, _BPW // 16, j2, 0)
        pltpu.sync_copy(woff_v.at[pl.ds(w * _BPW, _BPW)],
                        woff.at[w, pl.ds(b0, _BPW)])
        return 0
    lax.fori_loop(0, _NW, wr_loop, 0)
    plsc.subcore_barrier()

    pltpu.sync_copy(woff.at[wid], woff_v)

    def sc_loop(i, _):
        s = i * 16
        kv = keys_v[pl.ds(s, 16)]
        iv = idxs_v[pl.ds(s, 16)]
        d = digit_of(kv)
        cnt, msk = plsc.scan_count(d)
        bv = plsc.load_gather(woff_v, [d])
        pos = bv + cnt - 1
        row = i // 8
        col = (i % 8) * 16
        pos2_v[row, pl.ds(col, 16)] = pos
        posi_v[row, pl.ds(col, 16)] = pos + off
        kv2_v[row, pl.ds(col, 16)] = kv
        iv2_v[row, pl.ds(col, 16)] = iv
        ivi_v[row, pl.ds(col, 16)] = iv + off
        plsc.addupdate_scatter(woff_v, [d], cnt, mask=msk)
        return 0
    lax.fori_loop(0, _NV, sc_loop, 0)
    emit_scatter()


def _scatter_rows(dst_a, src_a, idx_a, dst_b, src_b, idx_b, sem):
    for j in range(_ROWS):
        pltpu.async_copy(src_a.at[j], dst_a.at[idx_a.at[j]], sem).wait()
        pltpu.async_copy(src_b.at[j], dst_b.at[idx_b.at[j]], sem).wait()


def _pass1_body(vxy, key1, idx1, hist, woff, smalls,
                keys_v, idxs_v, hist_v, woff_v, grid_v, cum_v, base_v,
                sm_v, buf_v, pos2_v, posi_v, kv2_v, iv2_v, ivi_v, sem):
    c = lax.axis_index("c")
    wid = lax.axis_index("s")
    off = c * _N
    base = wid * _CNK
    iota = lax.iota(jnp.int32, 16)
    refs = (keys_v, idxs_v, hist_v, woff_v, grid_v, cum_v, base_v,
            sm_v, buf_v, pos2_v, posi_v, kv2_v, iv2_v, ivi_v,
            hist.at[c], woff.at[c], smalls.at[c])

    def load_keys():
        pltpu.sync_copy(vxy.at[c, pl.ds(base, _CNK)], keys_v)

    def load_idxs():
        def il(i, _):
            idxs_v[pl.ds(i * 16, 16)] = base + i * 16 + iota
            return 0
        lax.fori_loop(0, _NV, il, 0)

    def emit_scatter():
        _scatter_rows(key1, kv2_v, posi_v, idx1, iv2_v, posi_v, sem)

    _pass_phases(lambda k: lax.bitwise_and(k, 8191),
                 load_keys, load_idxs, emit_scatter, refs, wid, off)


def _pass2_body(key1, idx1, permf, rankf, hist, woff, smalls,
                keys_v, idxs_v, hist_v, woff_v, grid_v, cum_v, base_v,
                sm_v, buf_v, pos2_v, posi_v, kv2_v, iv2_v, ivi_v, sem):
    c = lax.axis_index("c")
    wid = lax.axis_index("s")
    off = c * _N
    base = wid * _CNK
    refs = (keys_v, idxs_v, hist_v, woff_v, grid_v, cum_v, base_v,
            sm_v, buf_v, pos2_v, posi_v, kv2_v, iv2_v, ivi_v,
            hist.at[c], woff.at[c], smalls.at[c])

    def load_keys():
        pltpu.sync_copy(key1.at[pl.ds(off + base, _CNK)], keys_v)

    def load_idxs():
        pltpu.sync_copy(idx1.at[pl.ds(off + base, _CNK)], idxs_v)

    def emit_scatter():
        _scatter_rows(permf, iv2_v, posi_v, rankf, pos2_v, ivi_v, sem)

    _pass_phases(lambda k: lax.shift_right_logical(k, 13),
                 load_keys, load_idxs, emit_scatter, refs, wid, off)


_SCRATCH = None


def _mk_scratch():
    return [
        pltpu.VMEM((_CNK,), jnp.int32),
        pltpu.VMEM((_CNK,), jnp.int32),
        pltpu.VMEM((_BINS,), jnp.int32),
        pltpu.VMEM((_BINS,), jnp.int32),
        pltpu.VMEM((_NW, _BPW), jnp.int32),
        pltpu.VMEM((_BPW,), jnp.int32),
        pltpu.VMEM((_BPW,), jnp.int32),
        pltpu.VMEM((_NW, 16), jnp.int32),
        pltpu.VMEM((16,), jnp.int32),
        pltpu.VMEM((_ROWS, 128), jnp.int32),
        pltpu.VMEM((_ROWS, 128), jnp.int32),
        pltpu.VMEM((_ROWS, 128), jnp.int32),
        pltpu.VMEM((_ROWS, 128), jnp.int32),
        pltpu.VMEM((_ROWS, 128), jnp.int32),
        pltpu.SemaphoreType.DMA,
    ]


def _rank_pairs(vxy):
    mesh = plsc.VectorSubcoreMesh(core_axis_name="c", subcore_axis_name="s")
    grids = [
        jax.ShapeDtypeStruct((2, _NW, _BINS), jnp.int32),
        jax.ShapeDtypeStruct((2, _NW, _BINS), jnp.int32),
        jax.ShapeDtypeStruct((2, _NW, 16), jnp.int32),
    ]
    cp = pltpu.CompilerParams(needs_layout_passes=False)
    p1 = functools.partial(
        pl.kernel, mesh=mesh,
        out_type=[jax.ShapeDtypeStruct((2 * _N,), jnp.int32),
                  jax.ShapeDtypeStruct((2 * _N,), jnp.int32)] + grids,
        scratch_types=_mk_scratch(), compiler_params=cp)(_pass1_body)
    p2 = functools.partial(
        pl.kernel, mesh=mesh,
        out_type=[jax.ShapeDtypeStruct((2 * _N,), jnp.int32),
                  jax.ShapeDtypeStruct((2 * _N,), jnp.int32)] + grids,
        scratch_types=_mk_scratch(), compiler_params=cp)(_pass2_body)
    key1, idx1 = p1(vxy)[:2]
    permf, rankf = p2(key1, idx1)[:2]
    return permf[:_N], rankf[:_N], permf[_N:], rankf[_N:]


def kernel(x, coords, Wqkv_x, Wo_x, Wqkv_y, Wo_y):
    coords = coords.astype(jnp.int32)
    vx, vy = _window_keys(coords)
    perm_x, rank_x, perm_y, rank_y = _rank_pairs(jnp.stack([vx, vy]))
    cross = jnp.take(rank_x, perm_y)

    sorted1 = jnp.take(x, perm_x, axis=0)
    flat1 = _group_op(sorted1, Wqkv_x, Wo_x)
    sorted2 = jnp.take(flat1, cross, axis=0)
    flat2 = _group_op(sorted2, Wqkv_y, Wo_y)
    return jnp.take(flat2, rank_y, axis=0)


# speedup vs baseline: 1.1542x; 1.1542x over previous
import functools

import jax
import jax.numpy as jnp
from jax import lax
from jax.experimental import pallas as pl
from jax.experimental.pallas import tpu as pltpu
from jax.experimental.pallas import tpu_sc as plsc

_SPARSE_SHAPE = (32, 1000, 1000)
_WINDOW_SHAPE = (13, 13, 32)
_GROUP = 4096
_DIM = 128
_N = 65536
_NG = _N // _GROUP


def _window_keys(coords):
    import numpy as np
    sz, sy, sx = _SPARSE_SHAPE
    wx, wy, wz = _WINDOW_SHAPE
    max_x = int(np.ceil(sx / wx) + 1)
    max_y = int(np.ceil(sy / wy) + 1)
    max_z = int(np.ceil(sz / wz) + 1)
    m_per = max_x * max_y * max_z
    x = coords[:, 3]
    y = coords[:, 2]
    z = coords[:, 1]
    win_x = x // wx
    win_y = y // wy
    win_z = z // wz
    cix = x % wx
    ciy = y % wy
    ciz = z % wz
    bwx = coords[:, 0] * m_per + win_x * max_y * max_z + win_y * max_z + win_z
    bwy = coords[:, 0] * m_per + win_y * max_x * max_z + win_x * max_z + win_z
    wvol = wx * wy * wz
    vx = bwx * wvol + cix * (wy * wz) + ciy * wz + ciz
    vy = bwy * wvol + ciy * (wx * wz) + cix * wz + ciz
    return vx, vy


def _elu1(x):
    return jnp.where(x > 0, x + 1.0, jnp.exp(jnp.minimum(x, 0.0)))


def _group_body(x_ref, wqkv_ref, wo_ref, o_ref):
    x = x_ref[...]
    qkv = jnp.dot(x, wqkv_ref[...], preferred_element_type=jnp.float32)
    q = _elu1(qkv[:, :_DIM])
    k = _elu1(qkv[:, _DIM:2 * _DIM])
    v = qkv[:, 2 * _DIM:]
    vaug = jnp.concatenate([v, jnp.ones_like(v)], axis=1)
    s_aug = jax.lax.dot_general(k, vaug, (((0,), (0,)), ((), ())),
                                preferred_element_type=jnp.float32)
    nd = jnp.dot(q, s_aug, preferred_element_type=jnp.float32)
    out = nd[:, :_DIM] / (nd[:, _DIM:] + 1e-6)
    o_ref[...] = x + jnp.dot(out, wo_ref[...], preferred_element_type=jnp.float32)


@functools.partial(jax.jit, static_argnames=())
def _group_op(sorted_x, wqkv, wo):
    return pl.pallas_call(
        _group_body,
        grid=(_NG,),
        in_specs=[
            pl.BlockSpec((_GROUP, _DIM), lambda i: (i, 0)),
            pl.BlockSpec((_DIM, 3 * _DIM), lambda i: (0, 0)),
            pl.BlockSpec((_DIM, _DIM), lambda i: (0, 0)),
        ],
        out_specs=pl.BlockSpec((_GROUP, _DIM), lambda i: (i, 0)),
        out_shape=jax.ShapeDtypeStruct((_N, _DIM), jnp.float32),
    )(sorted_x, wqkv, wo)


_NW = 16
_CNK = _N // _NW
_NV = _CNK // 16
_BINS = 8192
_BPW = _BINS // _NW
_ROWS = _CNK // 128


def _pass_phases(digit_of, load_keys, load_idxs, emit_scatter,
                 refs, wid, off):
    (keys_v, idxs_v, hist_v, woff_v, grid_v, cum_v, base_v,
     sm_v, buf_v, pos2_v, posi_v, kv2_v, iv2_v, ivi_v,
     hist, woff, smalls) = refs
    iota = lax.iota(jnp.int32, 16)

    def zero(ref, nv):
        def zb(i, _):
            ref[pl.ds(i * 16, 16)] = jnp.zeros((16,), jnp.int32)
            return 0
        lax.fori_loop(0, nv, zb, 0)

    load_keys()
    load_idxs()
    zero(hist_v, _BINS // 16)

    def hist_loop(i, _):
        d = digit_of(keys_v[pl.ds(i * 16, 16)])
        cnt, msk = plsc.scan_count(d)
        plsc.addupdate_scatter(hist_v, [d], cnt, mask=msk)
        return 0
    lax.fori_loop(0, _NV, hist_loop, 0)
    pltpu.sync_copy(hist_v, hist.at[wid])
    plsc.subcore_barrier()

    b0 = wid * _BPW
    pltpu.sync_copy(hist.at[:, pl.ds(b0, _BPW)], grid_v)
    zero(cum_v, _BPW // 16)

    def w_loop(w, _):
        def j_loop(j, _):
            s = j * 16
            cv = cum_v[pl.ds(s, 16)]
            woff_v[pl.ds(w * _BPW + s, 16)] = cv
            cum_v[pl.ds(s, 16)] = cv + grid_v[w, pl.ds(s, 16)]
            return 0
        return lax.fori_loop(0, _BPW // 16, j_loop, 0)
    lax.fori_loop(0, _NW, w_loop, 0)

    def sum_loop(j, acc):
        return acc + jnp.sum(cum_v[pl.ds(j * 16, 16)])
    s_b = lax.fori_loop(0, _BPW // 16, sum_loop, jnp.int32(0))
    buf_v[...] = jnp.zeros((16,), jnp.int32) + s_b
    pltpu.sync_copy(buf_v, smalls.at[wid])
    plsc.subcore_barrier()
    pltpu.sync_copy(smalls, sm_v)
    svec = plsc.load_gather(sm_v, [iota, jnp.zeros((16,), jnp.int32)])
    csum = plsc.cumsum(svec)
    sb_excl = jnp.sum(jnp.where(iota == wid, csum - svec, 0))

    def bb_loop(j, carry):
        t = cum_v[pl.ds(j * 16, 16)]
        cs = plsc.cumsum(t)
        base_v[pl.ds(j * 16, 16)] = cs - t + carry
        return carry + jnp.sum(t)
    lax.fori_loop(0, _BPW // 16, bb_loop, sb_excl)

    def wr_loop(w, _):
        def j2(j, _):
            s = j * 16
            woff_v[pl.ds(w * _BPW + s, 16)] = (
                woff_v[pl.ds(w * _BPW + s, 16)] + base_v[pl.ds(s, 16)])
            return 0
        lax.fori_loop(0, _BPW // 16, j2, 0)
        pltpu.sync_copy(woff_v.at[pl.ds(w * _BPW, _BPW)],
                        woff.at[w, pl.ds(b0, _BPW)])
        return 0
    lax.fori_loop(0, _NW, wr_loop, 0)
    plsc.subcore_barrier()

    pltpu.sync_copy(woff.at[wid], woff_v)

    def sc_loop(i, _):
        s = i * 16
        kv = keys_v[pl.ds(s, 16)]
        iv = idxs_v[pl.ds(s, 16)]
        d = digit_of(kv)
        cnt, msk = plsc.scan_count(d)
        bv = plsc.load_gather(woff_v, [d])
        pos = bv + cnt - 1
        row = i // 8
        col = (i % 8) * 16
        pos2_v[row, pl.ds(col, 16)] = pos
        posi_v[row, pl.ds(col, 16)] = pos + off
        kv2_v[row, pl.ds(col, 16)] = kv
        iv2_v[row, pl.ds(col, 16)] = iv
        ivi_v[row, pl.ds(col, 16)] = iv + off
        plsc.addupdate_scatter(woff_v, [d], cnt, mask=msk)
        return 0
    lax.fori_loop(0, _NV, sc_loop, 0)
    emit_scatter()


def _scatter_rows(dst_a, src_a, idx_a, dst_b, src_b, idx_b, sem, sem_b):
    pend_a = pend_b = None
    for j in range(_ROWS):
        na = pltpu.async_copy(src_a.at[j], dst_a.at[idx_a.at[j]], sem)
        nb = pltpu.async_copy(src_b.at[j], dst_b.at[idx_b.at[j]], sem_b)
        if pend_a is not None:
            pend_a.wait()
            pend_b.wait()
        pend_a, pend_b = na, nb
    pend_a.wait()
    pend_b.wait()


def _pass1_body(vxy, key1, idx1, hist, woff, smalls,
                keys_v, idxs_v, hist_v, woff_v, grid_v, cum_v, base_v,
                sm_v, buf_v, pos2_v, posi_v, kv2_v, iv2_v, ivi_v, sem, sem_b):
    c = lax.axis_index("c")
    wid = lax.axis_index("s")
    off = c * _N
    base = wid * _CNK
    iota = lax.iota(jnp.int32, 16)
    refs = (keys_v, idxs_v, hist_v, woff_v, grid_v, cum_v, base_v,
            sm_v, buf_v, pos2_v, posi_v, kv2_v, iv2_v, ivi_v,
            hist.at[c], woff.at[c], smalls.at[c])

    def load_keys():
        pltpu.sync_copy(vxy.at[c, pl.ds(base, _CNK)], keys_v)

    def load_idxs():
        def il(i, _):
            idxs_v[pl.ds(i * 16, 16)] = base + i * 16 + iota
            return 0
        lax.fori_loop(0, _NV, il, 0)

    def emit_scatter():
        _scatter_rows(key1, kv2_v, posi_v, idx1, iv2_v, posi_v, sem, sem_b)

    _pass_phases(lambda k: lax.bitwise_and(k, 8191),
                 load_keys, load_idxs, emit_scatter, refs, wid, off)


def _pass2_body(key1, idx1, permf, rankf, hist, woff, smalls,
                keys_v, idxs_v, hist_v, woff_v, grid_v, cum_v, base_v,
                sm_v, buf_v, pos2_v, posi_v, kv2_v, iv2_v, ivi_v, sem, sem_b):
    c = lax.axis_index("c")
    wid = lax.axis_index("s")
    off = c * _N
    base = wid * _CNK
    refs = (keys_v, idxs_v, hist_v, woff_v, grid_v, cum_v, base_v,
            sm_v, buf_v, pos2_v, posi_v, kv2_v, iv2_v, ivi_v,
            hist.at[c], woff.at[c], smalls.at[c])

    def load_keys():
        pltpu.sync_copy(key1.at[pl.ds(off + base, _CNK)], keys_v)

    def load_idxs():
        pltpu.sync_copy(idx1.at[pl.ds(off + base, _CNK)], idxs_v)

    def emit_scatter():
        _scatter_rows(permf, iv2_v, posi_v, rankf, pos2_v, ivi_v, sem, sem_b)

    _pass_phases(lambda k: lax.shift_right_logical(k, 13),
                 load_keys, load_idxs, emit_scatter, refs, wid, off)


_SCRATCH = None


def _mk_scratch():
    return [
        pltpu.VMEM((_CNK,), jnp.int32),
        pltpu.VMEM((_CNK,), jnp.int32),
        pltpu.VMEM((_BINS,), jnp.int32),
        pltpu.VMEM((_BINS,), jnp.int32),
        pltpu.VMEM((_NW, _BPW), jnp.int32),
        pltpu.VMEM((_BPW,), jnp.int32),
        pltpu.VMEM((_BPW,), jnp.int32),
        pltpu.VMEM((_NW, 16), jnp.int32),
        pltpu.VMEM((16,), jnp.int32),
        pltpu.VMEM((_ROWS, 128), jnp.int32),
        pltpu.VMEM((_ROWS, 128), jnp.int32),
        pltpu.VMEM((_ROWS, 128), jnp.int32),
        pltpu.VMEM((_ROWS, 128), jnp.int32),
        pltpu.VMEM((_ROWS, 128), jnp.int32),
        pltpu.SemaphoreType.DMA,
        pltpu.SemaphoreType.DMA,
    ]


def _rank_pairs(vxy):
    mesh = plsc.VectorSubcoreMesh(core_axis_name="c", subcore_axis_name="s")
    grids = [
        jax.ShapeDtypeStruct((2, _NW, _BINS), jnp.int32),
        jax.ShapeDtypeStruct((2, _NW, _BINS), jnp.int32),
        jax.ShapeDtypeStruct((2, _NW, 16), jnp.int32),
    ]
    cp = pltpu.CompilerParams(needs_layout_passes=False)
    p1 = functools.partial(
        pl.kernel, mesh=mesh,
        out_type=[jax.ShapeDtypeStruct((2 * _N,), jnp.int32),
                  jax.ShapeDtypeStruct((2 * _N,), jnp.int32)] + grids,
        scratch_types=_mk_scratch(), compiler_params=cp)(_pass1_body)
    p2 = functools.partial(
        pl.kernel, mesh=mesh,
        out_type=[jax.ShapeDtypeStruct((2 * _N,), jnp.int32),
                  jax.ShapeDtypeStruct((2 * _N,), jnp.int32)] + grids,
        scratch_types=_mk_scratch(), compiler_params=cp)(_pass2_body)
    key1, idx1 = p1(vxy)[:2]
    permf, rankf = p2(key1, idx1)[:2]
    return permf[:_N], rankf[:_N], permf[_N:], rankf[_N:]


def kernel(x, coords, Wqkv_x, Wo_x, Wqkv_y, Wo_y):
    coords = coords.astype(jnp.int32)
    vx, vy = _window_keys(coords)
    perm_x, rank_x, perm_y, rank_y = _rank_pairs(jnp.stack([vx, vy]))
    cross = jnp.take(rank_x, perm_y)

    sorted1 = jnp.take(x, perm_x, axis=0)
    flat1 = _group_op(sorted1, Wqkv_x, Wo_x)
    sorted2 = jnp.take(flat1, cross, axis=0)
    flat2 = _group_op(sorted2, Wqkv_y, Wo_y)
    return jnp.take(flat2, rank_y, axis=0)


# speedup vs baseline: 1.3083x; 1.1335x over previous
import functools

import jax
import jax.numpy as jnp
from jax import lax
from jax.experimental import pallas as pl
from jax.experimental.pallas import tpu as pltpu
from jax.experimental.pallas import tpu_sc as plsc

_SPARSE_SHAPE = (32, 1000, 1000)
_WINDOW_SHAPE = (13, 13, 32)
_GROUP = 4096
_DIM = 128
_N = 65536
_NG = _N // _GROUP


def _window_keys(coords):
    import numpy as np
    sz, sy, sx = _SPARSE_SHAPE
    wx, wy, wz = _WINDOW_SHAPE
    max_x = int(np.ceil(sx / wx) + 1)
    max_y = int(np.ceil(sy / wy) + 1)
    max_z = int(np.ceil(sz / wz) + 1)
    m_per = max_x * max_y * max_z
    x = coords[:, 3]
    y = coords[:, 2]
    z = coords[:, 1]
    win_x = x // wx
    win_y = y // wy
    win_z = z // wz
    cix = x % wx
    ciy = y % wy
    ciz = z % wz
    bwx = coords[:, 0] * m_per + win_x * max_y * max_z + win_y * max_z + win_z
    bwy = coords[:, 0] * m_per + win_y * max_x * max_z + win_x * max_z + win_z
    wvol = wx * wy * wz
    vx = bwx * wvol + cix * (wy * wz) + ciy * wz + ciz
    vy = bwy * wvol + ciy * (wx * wz) + cix * wz + ciz
    return vx, vy


def _elu1(x):
    return jnp.where(x > 0, x + 1.0, jnp.exp(jnp.minimum(x, 0.0)))


def _group_body(x_ref, wqkv_ref, wo_ref, o_ref):
    x = x_ref[...]
    qkv = jnp.dot(x, wqkv_ref[...], preferred_element_type=jnp.float32)
    q = _elu1(qkv[:, :_DIM])
    k = _elu1(qkv[:, _DIM:2 * _DIM])
    v = qkv[:, 2 * _DIM:]
    vaug = jnp.concatenate([v, jnp.ones_like(v)], axis=1)
    s_aug = jax.lax.dot_general(k, vaug, (((0,), (0,)), ((), ())),
                                preferred_element_type=jnp.float32)
    nd = jnp.dot(q, s_aug, preferred_element_type=jnp.float32)
    out = nd[:, :_DIM] / (nd[:, _DIM:] + 1e-6)
    o_ref[...] = x + jnp.dot(out, wo_ref[...], preferred_element_type=jnp.float32)


@functools.partial(jax.jit, static_argnames=())
def _group_op(sorted_x, wqkv, wo):
    return pl.pallas_call(
        _group_body,
        grid=(_NG,),
        in_specs=[
            pl.BlockSpec((_GROUP, _DIM), lambda i: (i, 0)),
            pl.BlockSpec((_DIM, 3 * _DIM), lambda i: (0, 0)),
            pl.BlockSpec((_DIM, _DIM), lambda i: (0, 0)),
        ],
        out_specs=pl.BlockSpec((_GROUP, _DIM), lambda i: (i, 0)),
        out_shape=jax.ShapeDtypeStruct((_N, _DIM), jnp.float32),
    )(sorted_x, wqkv, wo)


_NW = 16
_CNK = _N // _NW
_NV = _CNK // 16
_BINS = 8192
_BPW = _BINS // _NW
_ROWS = _CNK // 128


def _pass_phases(digit_of, load_keys, load_idxs, emit_scatter,
                 refs, wid, off):
    (keys_v, idxs_v, hist_v, woff_v, grid_v, cum_v, base_v,
     sm_v, buf_v, pos2_v, posi_v, kv2_v, iv2_v, ivi_v,
     hist, woff, smalls) = refs
    iota = lax.iota(jnp.int32, 16)

    def zero(ref, nv):
        def zb(i, _):
            ref[pl.ds(i * 16, 16)] = jnp.zeros((16,), jnp.int32)
            return 0
        lax.fori_loop(0, nv, zb, 0)

    load_keys()
    load_idxs()
    zero(hist_v, _BINS // 16)

    def hist_loop(i, _):
        d = digit_of(keys_v[pl.ds(i * 16, 16)])
        cnt, msk = plsc.scan_count(d)
        plsc.addupdate_scatter(hist_v, [d], cnt, mask=msk)
        return 0
    lax.fori_loop(0, _NV, hist_loop, 0)
    pltpu.sync_copy(hist_v, hist.at[wid])
    plsc.subcore_barrier()

    b0 = wid * _BPW
    pltpu.sync_copy(hist.at[:, pl.ds(b0, _BPW)], grid_v)
    zero(cum_v, _BPW // 16)

    def w_loop(w, _):
        def j_loop(j, _):
            s = j * 16
            cv = cum_v[pl.ds(s, 16)]
            woff_v[pl.ds(w * _BPW + s, 16)] = cv
            cum_v[pl.ds(s, 16)] = cv + grid_v[w, pl.ds(s, 16)]
            return 0
        return lax.fori_loop(0, _BPW // 16, j_loop, 0)
    lax.fori_loop(0, _NW, w_loop, 0)

    def sum_loop(j, acc):
        return acc + jnp.sum(cum_v[pl.ds(j * 16, 16)])
    s_b = lax.fori_loop(0, _BPW // 16, sum_loop, jnp.int32(0))
    buf_v[...] = jnp.zeros((16,), jnp.int32) + s_b
    pltpu.sync_copy(buf_v, smalls.at[wid])
    plsc.subcore_barrier()
    pltpu.sync_copy(smalls, sm_v)
    svec = plsc.load_gather(sm_v, [iota, jnp.zeros((16,), jnp.int32)])
    csum = plsc.cumsum(svec)
    sb_excl = jnp.sum(jnp.where(iota == wid, csum - svec, 0))

    def bb_loop(j, carry):
        t = cum_v[pl.ds(j * 16, 16)]
        cs = plsc.cumsum(t)
        base_v[pl.ds(j * 16, 16)] = cs - t + carry
        return carry + jnp.sum(t)
    lax.fori_loop(0, _BPW // 16, bb_loop, sb_excl)

    def wr_loop(w, _):
        def j2(j, _):
            s = j * 16
            woff_v[pl.ds(w * _BPW + s, 16)] = (
                woff_v[pl.ds(w * _BPW + s, 16)] + base_v[pl.ds(s, 16)])
            return 0
        lax.fori_loop(0, _BPW // 16, j2, 0)
        pltpu.sync_copy(woff_v.at[pl.ds(w * _BPW, _BPW)],
                        woff.at[w, pl.ds(b0, _BPW)])
        return 0
    lax.fori_loop(0, _NW, wr_loop, 0)
    plsc.subcore_barrier()

    pltpu.sync_copy(woff.at[wid], woff_v)

    def sc_loop(i, _):
        s = i * 16
        kv = keys_v[pl.ds(s, 16)]
        iv = idxs_v[pl.ds(s, 16)]
        d = digit_of(kv)
        cnt, msk = plsc.scan_count(d)
        bv = plsc.load_gather(woff_v, [d])
        pos = bv + cnt - 1
        row = i // 8
        col = (i % 8) * 16
        pos2_v[row, pl.ds(col, 16)] = pos
        posi_v[row, pl.ds(col, 16)] = pos + off
        kv2_v[row, pl.ds(col, 16)] = lax.bitwise_or(
            lax.shift_left(lax.shift_right_logical(kv, 13), 16), iv)
        iv2_v[row, pl.ds(col, 16)] = iv
        ivi_v[row, pl.ds(col, 16)] = iv + off
        plsc.addupdate_scatter(woff_v, [d], cnt, mask=msk)
        return 0
    lax.fori_loop(0, _NV, sc_loop, 0)
    emit_scatter()


def _scatter_rows(dst_a, src_a, idx_a, dst_b, src_b, idx_b, sem, sem_b):
    pend_a = pend_b = None
    for j in range(_ROWS):
        na = pltpu.async_copy(src_a.at[j], dst_a.at[idx_a.at[j]], sem)
        nb = pltpu.async_copy(src_b.at[j], dst_b.at[idx_b.at[j]], sem_b)
        if pend_a is not None:
            pend_a.wait()
            pend_b.wait()
        pend_a, pend_b = na, nb
    pend_a.wait()
    pend_b.wait()


def _pass1_body(vxy, key1, hist, woff, smalls,
                keys_v, idxs_v, hist_v, woff_v, grid_v, cum_v, base_v,
                sm_v, buf_v, pos2_v, posi_v, kv2_v, iv2_v, ivi_v, sem, sem_b):
    c = lax.axis_index("c")
    wid = lax.axis_index("s")
    off = c * _N
    base = wid * _CNK
    iota = lax.iota(jnp.int32, 16)
    refs = (keys_v, idxs_v, hist_v, woff_v, grid_v, cum_v, base_v,
            sm_v, buf_v, pos2_v, posi_v, kv2_v, iv2_v, ivi_v,
            hist.at[c], woff.at[c], smalls.at[c])

    def load_keys():
        pltpu.sync_copy(vxy.at[c, pl.ds(base, _CNK)], keys_v)

    def load_idxs():
        def il(i, _):
            idxs_v[pl.ds(i * 16, 16)] = base + i * 16 + iota
            return 0
        lax.fori_loop(0, _NV, il, 0)

    def emit_scatter():
        pend = None
        for j in range(_ROWS):
            nxt = pltpu.async_copy(
                kv2_v.at[j], key1.at[posi_v.at[j]], sem if j % 2 else sem_b)
            if pend is not None:
                pend.wait()
            pend = nxt
        pend.wait()

    _pass_phases(lambda k: lax.bitwise_and(k, 8191),
                 load_keys, load_idxs, emit_scatter, refs, wid, off)


def _pass2_body(key1, permf, rankf, hist, woff, smalls,
                keys_v, idxs_v, hist_v, woff_v, grid_v, cum_v, base_v,
                sm_v, buf_v, pos2_v, posi_v, kv2_v, iv2_v, ivi_v, sem, sem_b):
    c = lax.axis_index("c")
    wid = lax.axis_index("s")
    off = c * _N
    base = wid * _CNK
    refs = (keys_v, idxs_v, hist_v, woff_v, grid_v, cum_v, base_v,
            sm_v, buf_v, pos2_v, posi_v, kv2_v, iv2_v, ivi_v,
            hist.at[c], woff.at[c], smalls.at[c])

    def load_keys():
        pltpu.sync_copy(key1.at[pl.ds(off + base, _CNK)], keys_v)

    def load_idxs():
        def il(i, _):
            s = i * 16
            idxs_v[pl.ds(s, 16)] = lax.bitwise_and(
                keys_v[pl.ds(s, 16)], 65535)
            return 0
        lax.fori_loop(0, _NV, il, 0)

    def emit_scatter():
        _scatter_rows(permf, iv2_v, posi_v, rankf, pos2_v, ivi_v, sem, sem_b)

    _pass_phases(lambda k: lax.shift_right_logical(k, 16),
                 load_keys, load_idxs, emit_scatter, refs, wid, off)


_SCRATCH = None


def _mk_scratch():
    return [
        pltpu.VMEM((_CNK,), jnp.int32),
        pltpu.VMEM((_CNK,), jnp.int32),
        pltpu.VMEM((_BINS,), jnp.int32),
        pltpu.VMEM((_BINS,), jnp.int32),
        pltpu.VMEM((_NW, _BPW), jnp.int32),
        pltpu.VMEM((_BPW,), jnp.int32),
        pltpu.VMEM((_BPW,), jnp.int32),
        pltpu.VMEM((_NW, 16), jnp.int32),
        pltpu.VMEM((16,), jnp.int32),
        pltpu.VMEM((_ROWS, 128), jnp.int32),
        pltpu.VMEM((_ROWS, 128), jnp.int32),
        pltpu.VMEM((_ROWS, 128), jnp.int32),
        pltpu.VMEM((_ROWS, 128), jnp.int32),
        pltpu.VMEM((_ROWS, 128), jnp.int32),
        pltpu.SemaphoreType.DMA,
        pltpu.SemaphoreType.DMA,
    ]


def _rank_pairs(vxy):
    mesh = plsc.VectorSubcoreMesh(core_axis_name="c", subcore_axis_name="s")
    grids = [
        jax.ShapeDtypeStruct((2, _NW, _BINS), jnp.int32),
        jax.ShapeDtypeStruct((2, _NW, _BINS), jnp.int32),
        jax.ShapeDtypeStruct((2, _NW, 16), jnp.int32),
    ]
    cp = pltpu.CompilerParams(needs_layout_passes=False)
    p1 = functools.partial(
        pl.kernel, mesh=mesh,
        out_type=[jax.ShapeDtypeStruct((2 * _N,), jnp.int32)] + grids,
        scratch_types=_mk_scratch(), compiler_params=cp)(_pass1_body)
    p2 = functools.partial(
        pl.kernel, mesh=mesh,
        out_type=[jax.ShapeDtypeStruct((2 * _N,), jnp.int32),
                  jax.ShapeDtypeStruct((2 * _N,), jnp.int32)] + grids,
        scratch_types=_mk_scratch(), compiler_params=cp)(_pass2_body)
    key1 = p1(vxy)[0]
    permf, rankf = p2(key1)[:2]
    return permf[:_N], rankf[:_N], permf[_N:], rankf[_N:]


def kernel(x, coords, Wqkv_x, Wo_x, Wqkv_y, Wo_y):
    coords = coords.astype(jnp.int32)
    vx, vy = _window_keys(coords)
    perm_x, rank_x, perm_y, rank_y = _rank_pairs(jnp.stack([vx, vy]))
    cross = jnp.take(rank_x, perm_y)

    sorted1 = jnp.take(x, perm_x, axis=0)
    flat1 = _group_op(sorted1, Wqkv_x, Wo_x)
    sorted2 = jnp.take(flat1, cross, axis=0)
    flat2 = _group_op(sorted2, Wqkv_y, Wo_y)
    return jnp.take(flat2, rank_y, axis=0)
